# baseline (speedup 1.0000x reference)
import functools

import jax
import jax.numpy as jnp
from jax import lax
from jax.experimental import pallas as pl
from jax.experimental.pallas import tpu as pltpu
from jax.experimental.pallas import tpu_sc as plsc

N = 10000
E = 320000
F_IN = 128
HID = 32
P = 12
NQ = P // 2
K3 = 3 * HID
K6 = 2 * K3

NC = 2
NS = 16
LANES = 128

RPT = 640
NPAD = NS * RPT
HSZ = NPAD // 2
HRT = HSZ // NS
ACR = HSZ + 16
JR = HSZ

EPT = E // NS
NW = 20
CH0 = NW * 8
CHD = (E + NC * NS * LANES - 1) // (NC * NS * LANES)
EPADD = NC * NS * CHD * LANES

LCAP = 184
JCH = 5

_mesh = plsc.VectorSubcoreMesh(core_axis_name="c", subcore_axis_name="s")


@functools.partial(
    pl.kernel,
    out_type=jax.ShapeDtypeStruct((NC, NPAD, 16), jnp.float32),
    mesh=_mesh,
    scratch_types=[
        pltpu.VMEM((CHD, LANES), jnp.int32),
        pltpu.VMEM((LANES, 16), jnp.float32),
        pltpu.VMEM((RPT, 16), jnp.float32),
        pltpu.SemaphoreType.DMA,
        pltpu.VMEM_SHARED((NPAD, 16), jnp.float32),
    ],
    compiler_params=pltpu.CompilerParams(use_tc_tiling_on_sc=False),
)
def _sc_deg(dst_hbm, ones_hbm, zero_hbm, out_hbm, idx_v, ones_v, zbuf_v, sem,
            acc_sh):
    c = lax.axis_index("c")
    s = lax.axis_index("s")
    pltpu.sync_copy(dst_hbm.at[c].at[s], idx_v)
    pltpu.sync_copy(ones_hbm, ones_v)
    pltpu.sync_copy(zero_hbm, zbuf_v)
    pltpu.sync_copy(zbuf_v, acc_sh.at[pl.ds(s * RPT, RPT)])
    plsc.subcore_barrier()

    MAXQ = 8

    def body(j, carry):
        pltpu.async_copy(ones_v, acc_sh.at[idx_v.at[j]], sem, add=True)

        @pl.when(j >= MAXQ)
        def _():
            pltpu.make_async_copy(ones_v, acc_sh.at[idx_v.at[0]], sem).wait()

        return carry

    lax.fori_loop(0, CHD, body, 0)
    for _ in range(MAXQ):
        pltpu.make_async_copy(ones_v, acc_sh.at[idx_v.at[0]], sem).wait()
    plsc.subcore_barrier()
    pltpu.sync_copy(acc_sh.at[pl.ds(s * RPT, RPT)],
                    out_hbm.at[c].at[pl.ds(s * RPT, RPT)])


@functools.partial(
    pl.kernel,
    out_type=[
        jax.ShapeDtypeStruct((NC, NS, LCAP, LANES), jnp.int32),
        jax.ShapeDtypeStruct((NC, NS, LCAP, LANES), jnp.int32),
        jax.ShapeDtypeStruct((NC, NS, 16), jnp.int32),
    ],
    mesh=_mesh,
    scratch_types=[
        pltpu.VMEM((8, LANES), jnp.int32),
        pltpu.VMEM((8, LANES), jnp.int32),
        pltpu.VMEM((160,), jnp.int32),
        pltpu.VMEM((160,), jnp.int32),
        pltpu.VMEM((16,), jnp.int32),
    ],
    compiler_params=pltpu.CompilerParams(use_tc_tiling_on_sc=False,
                                         needs_layout_passes=False),
)
def _sc_part(src_hbm, dst_hbm, ls_hbm, ld_hbm, cnt_hbm, win_s, win_d,
             stage_s, stage_d, cnt_v):
    c = lax.axis_index("c")
    s = lax.axis_index("s")
    lo = c * HSZ

    tpos = lax.iota(jnp.int32, 16)

    def flush(wp, op):
        @pl.when(wp >= LANES)
        def _():
            pltpu.sync_copy(stage_s.at[pl.ds(0, LANES)], ls_hbm.at[c, s, op])
            pltpu.sync_copy(stage_d.at[pl.ds(0, LANES)], ld_hbm.at[c, s, op])

        fl = (wp >= LANES).astype(jnp.int32)
        ts = stage_s[pl.ds(LANES, 16)]
        td = stage_d[pl.ds(LANES, 16)]
        off = (1 - fl) * LANES
        plsc.store_scatter(stage_s, [off + tpos], ts)
        plsc.store_scatter(stage_d, [off + tpos], td)
        return wp - LANES * fl, op + fl

    def window(w, carry):
        wp, op = carry
        pltpu.sync_copy(src_hbm.at[s].at[pl.ds(w * 8, 8)], win_s)
        pltpu.sync_copy(dst_hbm.at[s].at[pl.ds(w * 8, 8)], win_d)
        for j in range(8):
            for v in range(8):
                sv = win_s[j, pl.ds(v * 16, 16)]
                dv = win_d[j, pl.ds(v * 16, 16)]
                m = (dv >= lo) & (dv < lo + HSZ)
                csum = plsc.cumsum(m.astype(jnp.int32))
                pos = wp + csum - 1
                plsc.store_scatter(stage_d, [pos], dv - lo, mask=m)
                plsc.store_scatter(stage_s, [pos], sv, mask=m)
                wp = wp + csum[15]
                wp, op = flush(wp, op)
        return wp, op

    wp, op = lax.fori_loop(0, NW, window, (jnp.int32(0), jnp.int32(0)))
    jsrc = jnp.zeros((16,), jnp.int32)
    jdst = jnp.full((16,), JR, jnp.int32)

    jpos = lax.iota(jnp.int32, 16)

    def junk(t, carry):
        wp, op = carry
        plsc.store_scatter(stage_s, [wp + jpos], jsrc)
        plsc.store_scatter(stage_d, [wp + jpos], jdst)
        wp = wp + 16
        return flush(wp, op)

    wp, op = lax.fori_loop(0, JCH * 8, junk, (wp, op))
    for t in range(10):
        plsc.store_scatter(stage_s, [16 * t + jpos], jsrc)
        plsc.store_scatter(stage_d, [16 * t + jpos], jdst)

    def junkflush(t, carry):
        pltpu.sync_copy(stage_s.at[pl.ds(0, LANES)], ls_hbm.at[c, s, op + t])
        pltpu.sync_copy(stage_d.at[pl.ds(0, LANES)], ld_hbm.at[c, s, op + t])
        return carry

    lax.fori_loop(0, 16, junkflush, 0)
    cnt_v[...] = jnp.broadcast_to(op, (16,))
    pltpu.sync_copy(cnt_v, cnt_hbm.at[c, s])


@functools.partial(
    pl.kernel,
    out_type=jax.ShapeDtypeStruct((NQ, NPAD, K6), jnp.float32),
    mesh=_mesh,
    scratch_types=[
        pltpu.VMEM((16, LANES), jnp.int32),
        pltpu.VMEM((16, LANES), jnp.int32),
        pltpu.VMEM((LANES, K6), jnp.float32),
        pltpu.VMEM((LANES, K6), jnp.float32),
        pltpu.VMEM((16,), jnp.int32),
        pltpu.SemaphoreType.DMA,
        pltpu.SemaphoreType.DMA,
        pltpu.SemaphoreType.DMA,
        pltpu.SemaphoreType.DMA,
        pltpu.VMEM_SHARED((ACR, K6), jnp.float32),
    ],
    compiler_params=pltpu.CompilerParams(use_tc_tiling_on_sc=False),
)
def _sc_conv(m_hbm, ls_hbm, ld_hbm, cnt_hbm, out_hbm, src_w, dst_w, gb0, gb1,
             cnt_v, gs0, gs1, ss0, ss1, acc_sh):
    c = lax.axis_index("c")
    s = lax.axis_index("s")
    lo = c * HSZ
    lsrc = ls_hbm.at[c, s]
    ldst = ld_hbm.at[c, s]
    pltpu.sync_copy(cnt_hbm.at[c, s], cnt_v)
    op = cnt_v[pl.ds(0, 16)][0]
    nblocks = (op >> 3) + 1
    for q in range(NQ):
        for off, ln in ((0, LANES), (LANES, LANES), (2 * LANES, HRT - 256)):
            pltpu.sync_copy(m_hbm.at[q].at[pl.ds(lo + s * HRT + off, ln)],
                            gb0.at[pl.ds(0, ln)])
            pltpu.sync_copy(gb0.at[pl.ds(0, ln)],
                            acc_sh.at[pl.ds(s * HRT + off, ln)])
        plsc.subcore_barrier()

        pltpu.sync_copy(lsrc.at[pl.ds(0, 16)], src_w)
        pltpu.sync_copy(ldst.at[pl.ds(0, 16)], dst_w)
        pltpu.async_copy(m_hbm.at[q].at[src_w.at[0]], gb0, gs0)
        pltpu.async_copy(m_hbm.at[q].at[src_w.at[1]], gb1, gs1)

        def block(b, carry):
            base = 8 * b
            roff = base + 8
            woff = roff & 15
            pltpu.sync_copy(lsrc.at[pl.ds(roff, 8)], src_w.at[pl.ds(woff, 8)])
            pltpu.sync_copy(ldst.at[pl.ds(roff, 8)], dst_w.at[pl.ds(woff, 8)])
            for j2 in range(4):
                ch = base + 2 * j2
                pltpu.make_async_copy(m_hbm.at[q].at[src_w.at[ch & 15]], gb0,
                                      gs0).wait()
                pltpu.async_copy(gb0, acc_sh.at[dst_w.at[ch & 15]], ss0,
                                 add=True)
                pltpu.make_async_copy(m_hbm.at[q].at[src_w.at[(ch + 1) & 15]],
                                      gb1, gs1).wait()
                pltpu.async_copy(gb1, acc_sh.at[dst_w.at[(ch + 1) & 15]], ss1,
                                 add=True)
                pltpu.make_async_copy(gb0, acc_sh.at[dst_w.at[ch & 15]],
                                      ss0).wait()
                pltpu.async_copy(m_hbm.at[q].at[src_w.at[(ch + 2) & 15]], gb0,
                                 gs0)
                pltpu.make_async_copy(gb1, acc_sh.at[dst_w.at[(ch + 1) & 15]],
                                      ss1).wait()
                pltpu.async_copy(m_hbm.at[q].at[src_w.at[(ch + 3) & 15]], gb1,
                                 gs1)
            return carry

        lax.fori_loop(0, nblocks, block, 0)
        pltpu.make_async_copy(m_hbm.at[q].at[src_w.at[0]], gb0, gs0).wait()
        pltpu.make_async_copy(m_hbm.at[q].at[src_w.at[1]], gb1, gs1).wait()
        plsc.subcore_barrier()
        pltpu.sync_copy(acc_sh.at[pl.ds(s * HRT, HRT)],
                        out_hbm.at[q].at[pl.ds(lo + s * HRT, HRT)])


def _tc_proj_body(xt_ref, degs_ref, vcat_ref, m_ref):
    deg = degs_ref[0, :, 0] + degs_ref[1, :, 0] + 1.0
    dinv = lax.rsqrt(deg)[:, None]
    m0 = jnp.dot(xt_ref[0], vcat_ref[...], preferred_element_type=jnp.float32)
    m1 = jnp.dot(xt_ref[1], vcat_ref[...], preferred_element_type=jnp.float32)
    m_ref[0] = jnp.concatenate([m0 * dinv, m1 * dinv], axis=1)


def _tc_gru_body(s_ref, degs_ref, att_ref, lb_ref, lhb_ref, b2_ref,
                 wlin_ref, blin_ref, out_ref):
    nb = s_ref.shape[1]
    deg = degs_ref[0, :, 0] + degs_ref[1, :, 0] + 1.0
    dinv = lax.rsqrt(deg)[:, None]
    probs = jax.nn.softmax(att_ref[...])
    b2 = b2_ref[...][None, :]
    H = jnp.zeros((nb, HID), jnp.float32)
    acc = jnp.zeros((nb, HID), jnp.float32)
    for p in range(P):
        off = K3 * (p % 2)
        pre = s_ref[p // 2, :, off:off + K3] * dinv + b2
        HB = jnp.dot(H, lb_ref[...], preferred_element_type=jnp.float32)
        Z = jax.nn.sigmoid(pre[:, 0:HID] + HB[:, 0:HID])
        R = jax.nn.sigmoid(pre[:, HID:2 * HID] + HB[:, HID:2 * HID])
        Ht = jnp.tanh(pre[:, 2 * HID:3 * HID] +
                      jnp.dot(H * R, lhb_ref[...],
                              preferred_element_type=jnp.float32))
        H = Z * H + (1.0 - Z) * Ht
        acc = acc + probs[p] * H
    h = jnp.maximum(acc, 0.0)
    out_ref[...] = (jnp.dot(h, wlin_ref[...],
                            preferred_element_type=jnp.float32)
                    + blin_ref[...][None, :])


def kernel(x, att, Wz, bz, Lz, lbz, Wr, br, Lr, lbr, Wh, bh, Lh, lbh,
           Wlin, blin, edge_index):
    f32 = jnp.float32
    LzT, LzB = Lz[:HID], Lz[HID:]
    LrT, LrB = Lr[:HID], Lr[HID:]
    LhT, LhB = Lh[:HID], Lh[HID:]
    Vcat = jnp.concatenate([Wz @ LzT, Wr @ LrT, Wh @ LhT], axis=1)
    b2 = jnp.concatenate([bz @ LzT + lbz, br @ LrT + lbr, bh @ LhT + lbh])
    LB = jnp.concatenate([LzB, LrB], axis=1)

    xt = jnp.transpose(x, (2, 0, 1))
    xt = jnp.pad(xt, ((0, 0), (0, NPAD - N), (0, 0)))
    src = edge_index[0]
    dst = edge_index[1]
    padt = CH0 * LANES - EPT
    srcp = jnp.concatenate(
        [src.reshape(NS, EPT), jnp.zeros((NS, padt), jnp.int32)],
        axis=1).reshape(NS, CH0, LANES)
    dstp = jnp.concatenate(
        [dst.reshape(NS, EPT), jnp.full((NS, padt), N, jnp.int32)],
        axis=1).reshape(NS, CH0, LANES)
    dstd = jnp.concatenate(
        [dst, jnp.full((EPADD - E,), N, jnp.int32)]).reshape(NC, NS, CHD,
                                                             LANES)
    ones16 = jnp.ones((LANES, 16), f32)
    zrows = jnp.zeros((RPT, 16), f32)

    degs = _sc_deg(dstd, ones16, zrows)

    ls, ld, cnt = _sc_part(srcp, dstp)

    NB1 = RPT
    m_tab = pl.pallas_call(
        _tc_proj_body,
        grid=(NQ, NPAD // NB1),
        in_specs=[
            pl.BlockSpec((2, NB1, F_IN), lambda q, i: (q, i, 0)),
            pl.BlockSpec((NC, NB1, 16), lambda q, i: (0, i, 0)),
            pl.BlockSpec((F_IN, K3), lambda q, i: (0, 0)),
        ],
        out_specs=pl.BlockSpec((1, NB1, K6), lambda q, i: (q, i, 0)),
        out_shape=jax.ShapeDtypeStruct((NQ, NPAD, K6), f32),
    )(xt, degs, Vcat)

    s_tab = _sc_conv(m_tab, ls, ld, cnt)

    NB2 = RPT
    out = pl.pallas_call(
        _tc_gru_body,
        grid=(NPAD // NB2,),
        in_specs=[
            pl.BlockSpec((NQ, NB2, K6), lambda i: (0, i, 0)),
            pl.BlockSpec((NC, NB2, 16), lambda i: (0, i, 0)),
            pl.BlockSpec((P,), lambda i: (0,)),
            pl.BlockSpec((HID, 2 * HID), lambda i: (0, 0)),
            pl.BlockSpec((HID, HID), lambda i: (0, 0)),
            pl.BlockSpec((K3,), lambda i: (0,)),
            pl.BlockSpec((HID, P), lambda i: (0, 0)),
            pl.BlockSpec((P,), lambda i: (0,)),
        ],
        out_specs=pl.BlockSpec((NB2, P), lambda i: (i, 0)),
        out_shape=jax.ShapeDtypeStruct((NPAD, P), f32),
    )(s_tab, degs, att, LB, LhB, b2, Wlin, blin)
    return out[:N]


# speedup vs baseline: 3.8649x; 3.8649x over previous
import functools

import jax
import jax.numpy as jnp
from jax import lax
from jax.experimental import pallas as pl
from jax.experimental.pallas import tpu as pltpu
from jax.experimental.pallas import tpu_sc as plsc

N = 10000
E = 320000
F_IN = 128
HID = 32
P = 12
K3 = 3 * HID

NC = 2
NS = 16
LANES = 128

CH = (E // NS + LANES - 1) // LANES
CHP = CH + 1
CHX = CHP + 2
EPAD = NS * CHX * LANES
CHD = (E + NC * NS * LANES - 1) // (NC * NS * LANES)
EPADD = NC * NS * CHD * LANES

RPT = 632
NPAD = NS * RPT

_mesh = plsc.VectorSubcoreMesh(core_axis_name="c", subcore_axis_name="s")


@functools.partial(
    pl.kernel,
    out_type=jax.ShapeDtypeStruct((NC, NPAD, 16), jnp.float32),
    mesh=_mesh,
    scratch_types=[
        pltpu.VMEM((CHD, LANES), jnp.int32),
        pltpu.VMEM((LANES, 16), jnp.float32),
        pltpu.VMEM((NPAD // NS, 16), jnp.float32),
        pltpu.SemaphoreType.DMA,
        pltpu.VMEM_SHARED((NPAD, 16), jnp.float32),
    ],
    compiler_params=pltpu.CompilerParams(use_tc_tiling_on_sc=False),
)
def _sc_deg(dst_hbm, ones_hbm, zero_hbm, out_hbm, idx_v, ones_v, zbuf_v, sem,
            acc_sh):
    c = lax.axis_index("c")
    s = lax.axis_index("s")
    pltpu.sync_copy(dst_hbm.at[c].at[s], idx_v)
    pltpu.sync_copy(ones_hbm, ones_v)
    pltpu.sync_copy(zero_hbm, zbuf_v)
    zn = NPAD // NS
    pltpu.sync_copy(zbuf_v, acc_sh.at[pl.ds(s * zn, zn)])
    plsc.subcore_barrier()

    MAXQ = 8

    def body(j, carry):
        pltpu.async_copy(ones_v, acc_sh.at[idx_v.at[j]], sem, add=True)

        @pl.when(j >= MAXQ)
        def _():
            pltpu.make_async_copy(ones_v, acc_sh.at[idx_v.at[0]], sem).wait()

        return carry

    lax.fori_loop(0, CHD, body, 0)
    for _ in range(MAXQ):
        pltpu.make_async_copy(ones_v, acc_sh.at[idx_v.at[0]], sem).wait()
    plsc.subcore_barrier()
    pltpu.sync_copy(acc_sh.at[pl.ds(s * RPT, RPT)],
                    out_hbm.at[c].at[pl.ds(s * RPT, RPT)])


@functools.partial(
    pl.kernel,
    out_type=jax.ShapeDtypeStruct((P, NPAD, K3), jnp.float32),
    mesh=_mesh,
    scratch_types=[
        pltpu.VMEM((CHX, LANES), jnp.int32),
        pltpu.VMEM((CHX, LANES), jnp.int32),
        pltpu.VMEM((LANES, K3), jnp.float32),
        pltpu.VMEM((LANES, K3), jnp.float32),
        pltpu.SemaphoreType.DMA,
        pltpu.SemaphoreType.DMA,
        pltpu.VMEM_SHARED((NPAD, K3), jnp.float32),
    ],
    compiler_params=pltpu.CompilerParams(use_tc_tiling_on_sc=False),
)
def _sc_conv(m_hbm, src_hbm, dst_hbm, out_hbm, src_v, dst_v, gb0, gb1,
             gs0, gs1, acc_sh):
    c = lax.axis_index("c")
    s = lax.axis_index("s")
    pltpu.sync_copy(src_hbm.at[s], src_v)
    pltpu.sync_copy(dst_hbm.at[s], dst_v)
    for k in range(P // NC):
        p = k * NC + c
        for off, ln in ((0, 128), (128, 128), (256, 128), (384, 128),
                        (512, RPT - 512)):
            pltpu.sync_copy(m_hbm.at[p].at[pl.ds(s * RPT + off, ln)],
                            gb0.at[pl.ds(0, ln)])
            pltpu.sync_copy(gb0.at[pl.ds(0, ln)],
                            acc_sh.at[pl.ds(s * RPT + off, ln)])
        plsc.subcore_barrier()

        def step(j, carry):
            pltpu.async_copy(m_hbm.at[p].at[src_v.at[j]], gb0, gs0).wait()
            pltpu.sync_copy(gb0, acc_sh.at[dst_v.at[j]], add=True)
            return carry

        lax.fori_loop(0, CHP, step, 0)
        plsc.subcore_barrier()
        pltpu.sync_copy(acc_sh.at[pl.ds(s * RPT, RPT)],
                        out_hbm.at[p].at[pl.ds(s * RPT, RPT)])


def _tc_proj_body(xt_ref, degs_ref, vcat_ref, m_ref):
    deg = degs_ref[0, :, 0] + degs_ref[1, :, 0] + 1.0
    dinv = lax.rsqrt(deg)
    m = jnp.dot(xt_ref[0], vcat_ref[...], preferred_element_type=jnp.float32)
    m_ref[0] = m * dinv[:, None]


def _tc_gru_body(s_ref, degs_ref, att_ref, lb_ref, lhb_ref, b2_ref,
                 wlin_ref, blin_ref, out_ref):
    nb = s_ref.shape[1]
    deg = degs_ref[0, :, 0] + degs_ref[1, :, 0] + 1.0
    dinv = lax.rsqrt(deg)[:, None]
    probs = jax.nn.softmax(att_ref[...])
    b2 = b2_ref[...][None, :]
    H = jnp.zeros((nb, HID), jnp.float32)
    acc = jnp.zeros((nb, HID), jnp.float32)
    for p in range(P):
        pre = s_ref[p] * dinv + b2
        HB = jnp.dot(H, lb_ref[...], preferred_element_type=jnp.float32)
        Z = jax.nn.sigmoid(pre[:, 0:HID] + HB[:, 0:HID])
        R = jax.nn.sigmoid(pre[:, HID:2 * HID] + HB[:, HID:2 * HID])
        Ht = jnp.tanh(pre[:, 2 * HID:3 * HID] +
                      jnp.dot(H * R, lhb_ref[...],
                              preferred_element_type=jnp.float32))
        H = Z * H + (1.0 - Z) * Ht
        acc = acc + probs[p] * H
    h = jnp.maximum(acc, 0.0)
    out_ref[...] = (jnp.dot(h, wlin_ref[...],
                            preferred_element_type=jnp.float32)
                    + blin_ref[...][None, :])


def kernel(x, att, Wz, bz, Lz, lbz, Wr, br, Lr, lbr, Wh, bh, Lh, lbh,
           Wlin, blin, edge_index):
    f32 = jnp.float32
    LzT, LzB = Lz[:HID], Lz[HID:]
    LrT, LrB = Lr[:HID], Lr[HID:]
    LhT, LhB = Lh[:HID], Lh[HID:]
    Vcat = jnp.concatenate([Wz @ LzT, Wr @ LrT, Wh @ LhT], axis=1)
    b2 = jnp.concatenate([bz @ LzT + lbz, br @ LrT + lbr, bh @ LhT + lbh])
    LB = jnp.concatenate([LzB, LrB], axis=1)

    xt = jnp.transpose(x, (2, 0, 1))
    xt = jnp.pad(xt, ((0, 0), (0, NPAD - N), (0, 0)))
    src = edge_index[0]
    dst = edge_index[1]
    ept = E // NS
    padt = CHX * LANES - ept
    srcp = jnp.concatenate(
        [src.reshape(NS, ept), jnp.zeros((NS, padt), jnp.int32)],
        axis=1).reshape(NS, CHX, LANES)
    dstp = jnp.concatenate(
        [dst.reshape(NS, ept), jnp.full((NS, padt), N, jnp.int32)],
        axis=1).reshape(NS, CHX, LANES)
    dstd = jnp.concatenate(
        [dst, jnp.full((EPADD - E,), N, jnp.int32)]).reshape(NC, NS, CHD,
                                                             LANES)
    ones16 = jnp.ones((LANES, 16), f32)
    zrows = jnp.zeros((NPAD // NS, 16), f32)

    degs = _sc_deg(dstd, ones16, zrows)

    NB1 = RPT
    m_tab = pl.pallas_call(
        _tc_proj_body,
        grid=(P, NPAD // NB1),
        in_specs=[
            pl.BlockSpec((1, NB1, F_IN), lambda p, i: (p, i, 0)),
            pl.BlockSpec((NC, NB1, 16), lambda p, i: (0, i, 0)),
            pl.BlockSpec((F_IN, K3), lambda p, i: (0, 0)),
        ],
        out_specs=pl.BlockSpec((1, NB1, K3), lambda p, i: (p, i, 0)),
        out_shape=jax.ShapeDtypeStruct((P, NPAD, K3), f32),
    )(xt, degs, Vcat)

    s_tab = _sc_conv(m_tab, srcp, dstp)

    NB2 = RPT
    out = pl.pallas_call(
        _tc_gru_body,
        grid=(NPAD // NB2,),
        in_specs=[
            pl.BlockSpec((P, NB2, K3), lambda i: (0, i, 0)),
            pl.BlockSpec((NC, NB2, 16), lambda i: (0, i, 0)),
            pl.BlockSpec((P,), lambda i: (0,)),
            pl.BlockSpec((HID, 2 * HID), lambda i: (0, 0)),
            pl.BlockSpec((HID, HID), lambda i: (0, 0)),
            pl.BlockSpec((K3,), lambda i: (0,)),
            pl.BlockSpec((HID, P), lambda i: (0, 0)),
            pl.BlockSpec((P,), lambda i: (0,)),
        ],
        out_specs=pl.BlockSpec((NB2, P), lambda i: (i, 0)),
        out_shape=jax.ShapeDtypeStruct((NPAD, P), f32),
    )(s_tab, degs, att, LB, LhB, b2, Wlin, blin)
    return out[:N]


# speedup vs baseline: 3.8677x; 1.0007x over previous
import functools

import jax
import jax.numpy as jnp
from jax import lax
from jax.experimental import pallas as pl
from jax.experimental.pallas import tpu as pltpu
from jax.experimental.pallas import tpu_sc as plsc

N = 10000
E = 320000
F_IN = 128
HID = 32
P = 12
K3 = 3 * HID

NC = 2
NS = 16
LANES = 128

CH = (E // NS + LANES - 1) // LANES
CHP = CH + 1
CHX = CHP + 2
EPAD = NS * CHX * LANES
CHD = (E + NC * NS * LANES - 1) // (NC * NS * LANES)
EPADD = NC * NS * CHD * LANES

RPT = 632
NPAD = NS * RPT

_mesh = plsc.VectorSubcoreMesh(core_axis_name="c", subcore_axis_name="s")


@functools.partial(
    pl.kernel,
    out_type=jax.ShapeDtypeStruct((NC, NPAD, 16), jnp.float32),
    mesh=_mesh,
    scratch_types=[
        pltpu.VMEM((CHD, LANES), jnp.int32),
        pltpu.VMEM((LANES, 16), jnp.float32),
        pltpu.VMEM((NPAD // NS, 16), jnp.float32),
        pltpu.SemaphoreType.DMA,
        pltpu.VMEM_SHARED((NPAD, 16), jnp.float32),
    ],
    compiler_params=pltpu.CompilerParams(use_tc_tiling_on_sc=False),
)
def _sc_deg(dst_hbm, ones_hbm, zero_hbm, out_hbm, idx_v, ones_v, zbuf_v, sem,
            acc_sh):
    c = lax.axis_index("c")
    s = lax.axis_index("s")
    pltpu.sync_copy(dst_hbm.at[c].at[s], idx_v)
    pltpu.sync_copy(ones_hbm, ones_v)
    pltpu.sync_copy(zero_hbm, zbuf_v)
    zn = NPAD // NS
    pltpu.sync_copy(zbuf_v, acc_sh.at[pl.ds(s * zn, zn)])
    plsc.subcore_barrier()

    def body(j, carry):
        pltpu.sync_copy(ones_v, acc_sh.at[idx_v.at[j]], add=True)
        return carry

    lax.fori_loop(0, CHD, body, 0)
    plsc.subcore_barrier()
    pltpu.sync_copy(acc_sh.at[pl.ds(s * RPT, RPT)],
                    out_hbm.at[c].at[pl.ds(s * RPT, RPT)])


@functools.partial(
    pl.kernel,
    out_type=jax.ShapeDtypeStruct((P, NPAD, K3), jnp.float32),
    mesh=_mesh,
    scratch_types=[
        pltpu.VMEM((CHX, LANES), jnp.int32),
        pltpu.VMEM((CHX, LANES), jnp.int32),
        pltpu.VMEM((LANES, K3), jnp.float32),
        pltpu.VMEM((LANES, K3), jnp.float32),
        pltpu.SemaphoreType.DMA,
        pltpu.SemaphoreType.DMA,
        pltpu.VMEM_SHARED((NPAD, K3), jnp.float32),
    ],
    compiler_params=pltpu.CompilerParams(use_tc_tiling_on_sc=False),
)
def _sc_conv(m_hbm, src_hbm, dst_hbm, out_hbm, src_v, dst_v, gb0, gb1,
             gs0, gs1, acc_sh):
    c = lax.axis_index("c")
    s = lax.axis_index("s")
    pltpu.sync_copy(src_hbm.at[s], src_v)
    pltpu.sync_copy(dst_hbm.at[s], dst_v)
    for k in range(P // NC):
        p = k * NC + c
        for off, ln in ((0, 128), (128, 128), (256, 128), (384, 128),
                        (512, RPT - 512)):
            pltpu.sync_copy(m_hbm.at[p].at[pl.ds(s * RPT + off, ln)],
                            gb0.at[pl.ds(0, ln)])
            pltpu.sync_copy(gb0.at[pl.ds(0, ln)],
                            acc_sh.at[pl.ds(s * RPT + off, ln)])
        plsc.subcore_barrier()

        def step(j, carry):
            pltpu.async_copy(m_hbm.at[p].at[src_v.at[j]], gb0, gs0).wait()
            pltpu.sync_copy(gb0, acc_sh.at[dst_v.at[j]], add=True)
            return carry

        lax.fori_loop(0, CHP, step, 0)
        plsc.subcore_barrier()
        pltpu.sync_copy(acc_sh.at[pl.ds(s * RPT, RPT)],
                        out_hbm.at[p].at[pl.ds(s * RPT, RPT)])


def _tc_proj_body(xt_ref, degs_ref, vcat_ref, m_ref):
    deg = degs_ref[0, :, 0] + degs_ref[1, :, 0] + 1.0
    dinv = lax.rsqrt(deg)
    m = jnp.dot(xt_ref[0], vcat_ref[...], preferred_element_type=jnp.float32)
    m_ref[0] = m * dinv[:, None]


def _tc_gru_body(s_ref, degs_ref, att_ref, lb_ref, lhb_ref, b2_ref,
                 wlin_ref, blin_ref, out_ref):
    nb = s_ref.shape[1]
    deg = degs_ref[0, :, 0] + degs_ref[1, :, 0] + 1.0
    dinv = lax.rsqrt(deg)[:, None]
    probs = jax.nn.softmax(att_ref[...])
    b2 = b2_ref[...][None, :]
    H = jnp.zeros((nb, HID), jnp.float32)
    acc = jnp.zeros((nb, HID), jnp.float32)
    for p in range(P):
        pre = s_ref[p] * dinv + b2
        HB = jnp.dot(H, lb_ref[...], preferred_element_type=jnp.float32)
        Z = jax.nn.sigmoid(pre[:, 0:HID] + HB[:, 0:HID])
        R = jax.nn.sigmoid(pre[:, HID:2 * HID] + HB[:, HID:2 * HID])
        Ht = jnp.tanh(pre[:, 2 * HID:3 * HID] +
                      jnp.dot(H * R, lhb_ref[...],
                              preferred_element_type=jnp.float32))
        H = Z * H + (1.0 - Z) * Ht
        acc = acc + probs[p] * H
    h = jnp.maximum(acc, 0.0)
    out_ref[...] = (jnp.dot(h, wlin_ref[...],
                            preferred_element_type=jnp.float32)
                    + blin_ref[...][None, :])


def kernel(x, att, Wz, bz, Lz, lbz, Wr, br, Lr, lbr, Wh, bh, Lh, lbh,
           Wlin, blin, edge_index):
    f32 = jnp.float32
    LzT, LzB = Lz[:HID], Lz[HID:]
    LrT, LrB = Lr[:HID], Lr[HID:]
    LhT, LhB = Lh[:HID], Lh[HID:]
    Vcat = jnp.concatenate([Wz @ LzT, Wr @ LrT, Wh @ LhT], axis=1)
    b2 = jnp.concatenate([bz @ LzT + lbz, br @ LrT + lbr, bh @ LhT + lbh])
    LB = jnp.concatenate([LzB, LrB], axis=1)

    xt = jnp.transpose(x, (2, 0, 1))
    xt = jnp.pad(xt, ((0, 0), (0, NPAD - N), (0, 0)))
    src = edge_index[0]
    dst = edge_index[1]
    ept = E // NS
    padt = CHX * LANES - ept
    srcp = jnp.concatenate(
        [src.reshape(NS, ept), jnp.zeros((NS, padt), jnp.int32)],
        axis=1).reshape(NS, CHX, LANES)
    dstp = jnp.concatenate(
        [dst.reshape(NS, ept), jnp.full((NS, padt), N, jnp.int32)],
        axis=1).reshape(NS, CHX, LANES)
    dstd = jnp.concatenate(
        [dst, jnp.full((EPADD - E,), N, jnp.int32)]).reshape(NC, NS, CHD,
                                                             LANES)
    ones16 = jnp.ones((LANES, 16), f32)
    zrows = jnp.zeros((NPAD // NS, 16), f32)

    degs = _sc_deg(dstd, ones16, zrows)

    NB1 = RPT
    m_tab = pl.pallas_call(
        _tc_proj_body,
        grid=(P, NPAD // NB1),
        in_specs=[
            pl.BlockSpec((1, NB1, F_IN), lambda p, i: (p, i, 0)),
            pl.BlockSpec((NC, NB1, 16), lambda p, i: (0, i, 0)),
            pl.BlockSpec((F_IN, K3), lambda p, i: (0, 0)),
        ],
        out_specs=pl.BlockSpec((1, NB1, K3), lambda p, i: (p, i, 0)),
        out_shape=jax.ShapeDtypeStruct((P, NPAD, K3), f32),
    )(xt, degs, Vcat)

    s_tab = _sc_conv(m_tab, srcp, dstp)

    NB2 = RPT
    out = pl.pallas_call(
        _tc_gru_body,
        grid=(NPAD // NB2,),
        in_specs=[
            pl.BlockSpec((P, NB2, K3), lambda i: (0, i, 0)),
            pl.BlockSpec((NC, NB2, 16), lambda i: (0, i, 0)),
            pl.BlockSpec((P,), lambda i: (0,)),
            pl.BlockSpec((HID, 2 * HID), lambda i: (0, 0)),
            pl.BlockSpec((HID, HID), lambda i: (0, 0)),
            pl.BlockSpec((K3,), lambda i: (0,)),
            pl.BlockSpec((HID, P), lambda i: (0, 0)),
            pl.BlockSpec((P,), lambda i: (0,)),
        ],
        out_specs=pl.BlockSpec((NB2, P), lambda i: (i, 0)),
        out_shape=jax.ShapeDtypeStruct((NPAD, P), f32),
    )(s_tab, degs, att, LB, LhB, b2, Wlin, blin)
    return out[:N]
